# baseline (device time: 123132 ns/iter reference)
import jax
import jax.numpy as jnp
from jax import lax
from jax.experimental import pallas as pl
from jax.experimental.pallas import tpu as pltpu

S, D, H, Dh, Dr = 1024, 2048, 16, 128, 32
DC_SH = 128
DC = 2 * DC_SH
SCALE = (Dh + Dr) ** -0.5
BF = jnp.bfloat16


def _proj_exchange_body(x_ref, wdkv_ref, wuk_ref, wuv_ref,
                        wqr_ref, wkr_ref,
                        k_ref, v_ref, qrt_ref, kr_ref,
                        c_all, wuk_all, wuv_all, qrt_tmp, kr_tmp,
                        send_sems, recv_sems, out_sems):
    my_x = lax.axis_index("x")
    my_y = lax.axis_index("y")
    my_z = lax.axis_index("z")
    peer = (my_x, my_y, 1 - my_z)
    x = x_ref[0]

    barrier = pltpu.get_barrier_semaphore()
    pl.semaphore_signal(barrier, inc=1, device_id=peer,
                        device_id_type=pl.DeviceIdType.MESH)
    pl.semaphore_wait(barrier, 1)

    wuk_all[:DC_SH, :] = wuk_ref[...].astype(BF)
    rdma_wuk = pltpu.make_async_remote_copy(
        src_ref=wuk_all.at[:DC_SH, :], dst_ref=wuk_all.at[DC_SH:, :],
        send_sem=send_sems.at[0], recv_sem=recv_sems.at[0],
        device_id=peer, device_id_type=pl.DeviceIdType.MESH)
    rdma_wuk.start()
    wuv_all[:DC_SH, :] = wuv_ref[...].astype(BF)
    rdma_wuv = pltpu.make_async_remote_copy(
        src_ref=wuv_all.at[:DC_SH, :], dst_ref=wuv_all.at[DC_SH:, :],
        send_sem=send_sems.at[1], recv_sem=recv_sems.at[1],
        device_id=peer, device_id_type=pl.DeviceIdType.MESH)
    rdma_wuv.start()

    c_all[:, :DC_SH] = jnp.dot(
        x, wdkv_ref[...], preferred_element_type=jnp.float32).astype(BF)
    rdma_c = pltpu.make_async_remote_copy(
        src_ref=c_all.at[:, :DC_SH], dst_ref=c_all.at[:, DC_SH:],
        send_sem=send_sems.at[2], recv_sem=recv_sems.at[2],
        device_id=peer, device_id_type=pl.DeviceIdType.MESH)
    rdma_c.start()

    qrt_tmp[...] = (lax.dot_general(
        wqr_ref[...], x, (((0,), (1,)), ((), ())),
        preferred_element_type=jnp.float32) * SCALE).astype(BF)
    kr_tmp[...] = jnp.dot(x, wkr_ref[...],
                          preferred_element_type=jnp.float32).astype(BF)
    cp_qrt = pltpu.make_async_copy(qrt_tmp, qrt_ref, out_sems.at[0])
    cp_qrt.start()
    cp_kr = pltpu.make_async_copy(kr_tmp, kr_ref, out_sems.at[1])
    cp_kr.start()

    rdma_wuk.wait()
    rdma_c.wait()
    k_ref[...] = jnp.dot(c_all[...], wuk_all[...],
                         preferred_element_type=jnp.float32).astype(BF)
    rdma_wuv.wait()
    v_ref[...] = jnp.dot(c_all[...], wuv_all[...],
                         preferred_element_type=jnp.float32).astype(BF)

    cp_qrt.wait()
    cp_kr.wait()


def _proj_exchange(x, Wdkv, Wuk, Wuv, Wqr, Wkr):
    return pl.pallas_call(
        _proj_exchange_body,
        out_shape=(jax.ShapeDtypeStruct((S, D), BF),
                   jax.ShapeDtypeStruct((S, D), BF),
                   jax.ShapeDtypeStruct((H * Dr, S), BF),
                   jax.ShapeDtypeStruct((S, Dr), BF)),
        in_specs=[pl.BlockSpec(memory_space=pltpu.VMEM)] * 6,
        out_specs=(pl.BlockSpec(memory_space=pltpu.VMEM),
                   pl.BlockSpec(memory_space=pltpu.VMEM),
                   pl.BlockSpec(memory_space=pl.ANY),
                   pl.BlockSpec(memory_space=pl.ANY)),
        scratch_shapes=[
            pltpu.VMEM((S, DC), BF),
            pltpu.VMEM((DC, D), BF),
            pltpu.VMEM((DC, D), BF),
            pltpu.VMEM((H * Dr, S), BF),
            pltpu.VMEM((S, Dr), BF),
            pltpu.SemaphoreType.DMA((3,)),
            pltpu.SemaphoreType.DMA((3,)),
            pltpu.SemaphoreType.DMA((2,)),
        ],
        compiler_params=pltpu.CompilerParams(
            collective_id=0,
            vmem_limit_bytes=100 * 1024 * 1024,
        ),
    )(x, Wdkv, Wuk, Wuv, Wqr, Wkr)


def _qproj_body(x_ref, wq_ref, q_ref):
    xb = x_ref[0].astype(BF)
    wb = wq_ref[...].astype(BF)
    q_ref[...] = (jnp.dot(xb, wb, preferred_element_type=jnp.float32)
                  * SCALE).astype(BF)


def _qproj(x, Wq):
    return pl.pallas_call(
        _qproj_body,
        out_shape=jax.ShapeDtypeStruct((S, D), BF),
        in_specs=[pl.BlockSpec(memory_space=pltpu.VMEM)] * 2,
        out_specs=pl.BlockSpec(memory_space=pltpu.VMEM),
    )(x, Wq)


def _attn_body(q_ref, k_ref, v_ref, qrt_ref, kr_ref, o_ref):
    s = lax.dot_general(q_ref[...], k_ref[...],
                        (((1,), (1,)), ((), ())),
                        preferred_element_type=jnp.float32)
    s += lax.dot_general(qrt_ref[...], kr_ref[...],
                         (((0,), (1,)), ((), ())),
                         preferred_element_type=jnp.float32)
    p = jnp.exp(s)
    denom = jnp.sum(p, axis=1, keepdims=True)
    o = jnp.dot(p.astype(BF), v_ref[...],
                preferred_element_type=jnp.float32)
    o_ref[...] = (o / denom).astype(BF)


def _attn(q, k, v, qrt, kr):
    return pl.pallas_call(
        _attn_body,
        grid=(H,),
        out_shape=jax.ShapeDtypeStruct((S, D), BF),
        in_specs=[
            pl.BlockSpec((S, Dh), lambda h: (0, h)),
            pl.BlockSpec((S, Dh), lambda h: (0, h)),
            pl.BlockSpec((S, Dh), lambda h: (0, h)),
            pl.BlockSpec((Dr, S), lambda h: (h, 0)),
            pl.BlockSpec((S, Dr), lambda h: (0, 0)),
        ],
        out_specs=pl.BlockSpec((S, Dh), lambda h: (0, h)),
        compiler_params=pltpu.CompilerParams(
            dimension_semantics=("arbitrary",)),
    )(q, k, v, qrt, kr)


def _oproj_body(o_ref, wo_ref, out_ref):
    wb = wo_ref[...].astype(BF)
    out_ref[0] = jnp.dot(o_ref[...], wb,
                         preferred_element_type=jnp.float32)


def _oproj(o, Wo):
    return pl.pallas_call(
        _oproj_body,
        out_shape=jax.ShapeDtypeStruct((1, S, D), jnp.float32),
        in_specs=[pl.BlockSpec(memory_space=pltpu.VMEM)] * 2,
        out_specs=pl.BlockSpec(memory_space=pltpu.VMEM),
    )(o, Wo)


def kernel(x, Wdkv, Wuk, Wuv, Wq, Wqr, Wkr, Wo):
    k, v, qrt, kr = _proj_exchange(x, Wdkv, Wuk, Wuv, Wqr, Wkr)
    q = _qproj(x, Wq)
    o = _attn(q, k, v, qrt, kr)
    return _oproj(o, Wo)


# device time: 82032 ns/iter; 1.5010x vs baseline; 1.5010x over previous
import jax
import jax.numpy as jnp
from jax import lax
from jax.experimental import pallas as pl
from jax.experimental.pallas import tpu as pltpu

S, D, H, Dh, Dr = 1024, 2048, 16, 128, 32
DC_SH = 128
DC = 2 * DC_SH
SCALE = (Dh + Dr) ** -0.5
BF = jnp.bfloat16

N_CHUNK = 2
CS = S // N_CHUNK


def _mla_body(x_ref, wdkv_ref, wuk_ref, wuv_ref, wq_ref, wqr_ref, wkrt_ref,
              o_ref,
              c_all, wuk_all, wuv_all, k_scr, v_scr, q_scr, qrt_scr, krt_scr,
              send_sems, recv_sems):
    my_x = lax.axis_index("x")
    my_y = lax.axis_index("y")
    my_z = lax.axis_index("z")
    peer = (my_x, my_y, 1 - my_z)
    x = x_ref[0]

    barrier = pltpu.get_barrier_semaphore()
    pl.semaphore_signal(barrier, inc=1, device_id=peer,
                        device_id_type=pl.DeviceIdType.MESH)
    pl.semaphore_wait(barrier, 1)

    wuk_all[:DC_SH, :] = wuk_ref[...].astype(BF)
    rdma_wuk = pltpu.make_async_remote_copy(
        src_ref=wuk_all.at[:DC_SH, :], dst_ref=wuk_all.at[DC_SH:, :],
        send_sem=send_sems.at[0], recv_sem=recv_sems.at[0],
        device_id=peer, device_id_type=pl.DeviceIdType.MESH)
    rdma_wuk.start()
    wuv_all[:DC_SH, :] = wuv_ref[...].astype(BF)
    rdma_wuv = pltpu.make_async_remote_copy(
        src_ref=wuv_all.at[:DC_SH, :], dst_ref=wuv_all.at[DC_SH:, :],
        send_sem=send_sems.at[1], recv_sem=recv_sems.at[1],
        device_id=peer, device_id_type=pl.DeviceIdType.MESH)
    rdma_wuv.start()

    c_all[:, :DC_SH] = jnp.dot(
        x, wdkv_ref[...], preferred_element_type=jnp.float32).astype(BF)
    rdma_c = pltpu.make_async_remote_copy(
        src_ref=c_all.at[:, :DC_SH], dst_ref=c_all.at[:, DC_SH:],
        send_sem=send_sems.at[2], recv_sem=recv_sems.at[2],
        device_id=peer, device_id_type=pl.DeviceIdType.MESH)
    rdma_c.start()

    q_scr[...] = (jnp.dot(x.astype(BF), wq_ref[...].astype(BF),
                          preferred_element_type=jnp.float32)
                  * SCALE).astype(BF)
    qrt_scr[...] = (lax.dot_general(
        wqr_ref[...], x, (((0,), (1,)), ((), ())),
        preferred_element_type=jnp.float32) * SCALE).astype(BF)
    krt_scr[...] = lax.dot_general(
        wkrt_ref[...], x, (((1,), (1,)), ((), ())),
        preferred_element_type=jnp.float32).astype(BF)

    rdma_wuk.wait()
    rdma_c.wait()
    k_scr[...] = jnp.dot(c_all[...], wuk_all[...],
                         preferred_element_type=jnp.float32).astype(BF)
    rdma_wuv.wait()
    v_scr[...] = jnp.dot(c_all[...], wuv_all[...],
                         preferred_element_type=jnp.float32).astype(BF)

    for h in range(H):
        hc = slice(h * Dh, (h + 1) * Dh)
        q = q_scr[:, hc]
        qrt = qrt_scr[h * Dr:(h + 1) * Dr, :]
        o = jnp.zeros((S, Dh), jnp.float32)
        denom = jnp.zeros((S, 1), jnp.float32)
        for j in range(N_CHUNK):
            rc = slice(j * CS, (j + 1) * CS)
            sj = lax.dot_general(q, k_scr[rc, hc],
                                 (((1,), (1,)), ((), ())),
                                 preferred_element_type=jnp.float32)
            sj += lax.dot_general(qrt, krt_scr[:, rc],
                                  (((0,), (0,)), ((), ())),
                                  preferred_element_type=jnp.float32)
            pj = jnp.exp(sj)
            denom += jnp.sum(pj, axis=1, keepdims=True)
            o += jnp.dot(pj.astype(BF), v_scr[rc, hc],
                         preferred_element_type=jnp.float32)
        o_ref[:, hc] = (o / denom).astype(BF)


def _mla(x, Wdkv, Wuk, Wuv, Wq, Wqr, Wkr):
    return pl.pallas_call(
        _mla_body,
        out_shape=jax.ShapeDtypeStruct((S, D), BF),
        in_specs=[pl.BlockSpec(memory_space=pltpu.VMEM)] * 7,
        out_specs=pl.BlockSpec(memory_space=pltpu.VMEM),
        scratch_shapes=[
            pltpu.VMEM((S, DC), BF),
            pltpu.VMEM((DC, D), BF),
            pltpu.VMEM((DC, D), BF),
            pltpu.VMEM((S, D), BF),
            pltpu.VMEM((S, D), BF),
            pltpu.VMEM((S, D), BF),
            pltpu.VMEM((H * Dr, S), BF),
            pltpu.VMEM((Dr, S), BF),
            pltpu.SemaphoreType.DMA((3,)),
            pltpu.SemaphoreType.DMA((3,)),
        ],
        compiler_params=pltpu.CompilerParams(
            collective_id=0,
            vmem_limit_bytes=100 * 1024 * 1024,
        ),
    )(x, Wdkv, Wuk, Wuv, Wq, Wqr, Wkr)


N_OCHUNK = 4
OCS = D // N_OCHUNK


def _oproj_body(o_ref, wo_ref, out_ref):
    out_ref[0] = jnp.dot(o_ref[...], wo_ref[...],
                         preferred_element_type=jnp.float32)


def _oproj(o, Wo):
    return pl.pallas_call(
        _oproj_body,
        grid=(N_OCHUNK,),
        out_shape=jax.ShapeDtypeStruct((1, S, D), jnp.float32),
        in_specs=[
            pl.BlockSpec((S, D), lambda j: (0, 0)),
            pl.BlockSpec((D, OCS), lambda j: (0, j)),
        ],
        out_specs=pl.BlockSpec((1, S, OCS), lambda j: (0, 0, j)),
        compiler_params=pltpu.CompilerParams(
            dimension_semantics=("arbitrary",)),
    )(o, Wo)


def kernel(x, Wdkv, Wuk, Wuv, Wq, Wqr, Wkr, Wo):
    o = _mla(x, Wdkv, Wuk, Wuv, Wq, Wqr, Wkr.T)
    return _oproj(o, Wo.astype(BF))
